# baseline (device time: 23831 ns/iter reference)
import jax
import jax.numpy as jnp
from jax import lax
from jax.experimental import pallas as pl
from jax.experimental.pallas import tpu as pltpu

EPS = 1e-5
Y_SIZE = 2


def kernel(x, gamma):
    m, n_loc = x.shape
    n_global = Y_SIZE * n_loc
    gamma2d = gamma.reshape(1, n_loc)

    def body(x_ref, g_ref, out_ref, psum_ref, recv_ref, send_sem, recv_sem):
        my_x = lax.axis_index("x")
        my_y = lax.axis_index("y")
        peer = (my_x, 1 - my_y)

        xf = x_ref[:, :].astype(jnp.float32)
        psum_ref[:, :] = jnp.sum(xf * xf, axis=1, keepdims=True)

        barrier_sem = pltpu.get_barrier_semaphore()
        pl.semaphore_signal(
            barrier_sem, inc=1, device_id=peer,
            device_id_type=pl.DeviceIdType.MESH,
        )
        pl.semaphore_wait(barrier_sem, 1)

        rdma = pltpu.make_async_remote_copy(
            src_ref=psum_ref,
            dst_ref=recv_ref,
            send_sem=send_sem,
            recv_sem=recv_sem,
            device_id=peer,
            device_id_type=pl.DeviceIdType.MESH,
        )
        rdma.start()
        rdma.wait()

        total = psum_ref[:, :] + recv_ref[:, :]
        inv_rms = lax.rsqrt(total / n_global + EPS)
        gf = g_ref[:, :].astype(jnp.float32)
        out_ref[:, :] = (xf * gf * inv_rms).astype(out_ref.dtype)

    return pl.pallas_call(
        body,
        out_shape=jax.ShapeDtypeStruct((m, n_loc), x.dtype),
        in_specs=[
            pl.BlockSpec(memory_space=pltpu.VMEM),
            pl.BlockSpec(memory_space=pltpu.VMEM),
        ],
        out_specs=pl.BlockSpec(memory_space=pltpu.VMEM),
        scratch_shapes=[
            pltpu.VMEM((m, 1), jnp.float32),
            pltpu.VMEM((m, 1), jnp.float32),
            pltpu.SemaphoreType.DMA,
            pltpu.SemaphoreType.DMA,
        ],
        compiler_params=pltpu.CompilerParams(collective_id=0),
    )(x, gamma2d)


# device time: 22111 ns/iter; 1.0778x vs baseline; 1.0778x over previous
import jax
import jax.numpy as jnp
from jax import lax
from jax.experimental import pallas as pl
from jax.experimental.pallas import tpu as pltpu

EPS = 1e-5
Y_SIZE = 2
N_CHUNKS = 8


def kernel(x, gamma):
    m, n_loc = x.shape
    n_global = Y_SIZE * n_loc
    chunk_m = m // N_CHUNKS
    gamma2d = gamma.reshape(1, n_loc)

    def body(x_hbm, g_ref, out_hbm, xv, outv, psum_ref, recv_ref,
             load_sems, store_sems, send_sems, recv_sems):
        my_x = lax.axis_index("x")
        my_y = lax.axis_index("y")
        peer = (my_x, 1 - my_y)

        rows = [pl.ds(j * chunk_m, chunk_m) for j in range(N_CHUNKS)]

        loads = []
        for j in range(N_CHUNKS):
            cp = pltpu.make_async_copy(
                x_hbm.at[rows[j], :], xv.at[rows[j], :], load_sems.at[j]
            )
            cp.start()
            loads.append(cp)

        barrier_sem = pltpu.get_barrier_semaphore()
        pl.semaphore_signal(
            barrier_sem, inc=1, device_id=peer,
            device_id_type=pl.DeviceIdType.MESH,
        )
        pl.semaphore_wait(barrier_sem, 1)

        gf = g_ref[:, :].astype(jnp.float32)

        rdmas = []
        for j in range(N_CHUNKS):
            loads[j].wait()
            xf = xv[rows[j], :]
            psum_ref[rows[j], :] = jnp.sum(xf * xf, axis=1, keepdims=True)
            rdma = pltpu.make_async_remote_copy(
                src_ref=psum_ref.at[rows[j], :],
                dst_ref=recv_ref.at[rows[j], :],
                send_sem=send_sems.at[j],
                recv_sem=recv_sems.at[j],
                device_id=peer,
                device_id_type=pl.DeviceIdType.MESH,
            )
            rdma.start()
            rdmas.append(rdma)

        stores = []
        for j in range(N_CHUNKS):
            rdmas[j].wait_recv()
            total = psum_ref[rows[j], :] + recv_ref[rows[j], :]
            inv_rms = lax.rsqrt(total / n_global + EPS)
            outv[rows[j], :] = (xv[rows[j], :] * gf * inv_rms).astype(
                outv.dtype
            )
            st = pltpu.make_async_copy(
                outv.at[rows[j], :], out_hbm.at[rows[j], :], store_sems.at[j]
            )
            st.start()
            stores.append(st)

        for j in range(N_CHUNKS):
            rdmas[j].wait_send()
            stores[j].wait()

    return pl.pallas_call(
        body,
        out_shape=jax.ShapeDtypeStruct((m, n_loc), jnp.bfloat16),
        in_specs=[
            pl.BlockSpec(memory_space=pl.ANY),
            pl.BlockSpec(memory_space=pltpu.VMEM),
        ],
        out_specs=pl.BlockSpec(memory_space=pl.ANY),
        scratch_shapes=[
            pltpu.VMEM((m, n_loc), jnp.float32),
            pltpu.VMEM((m, n_loc), jnp.bfloat16),
            pltpu.VMEM((m, 1), jnp.float32),
            pltpu.VMEM((m, 1), jnp.float32),
            pltpu.SemaphoreType.DMA((N_CHUNKS,)),
            pltpu.SemaphoreType.DMA((N_CHUNKS,)),
            pltpu.SemaphoreType.DMA((N_CHUNKS,)),
            pltpu.SemaphoreType.DMA((N_CHUNKS,)),
        ],
        compiler_params=pltpu.CompilerParams(collective_id=0),
    )(x, gamma2d)
